# baseline (speedup 1.0000x reference)
import functools

import jax
import jax.numpy as jnp
from jax.experimental import pallas as pl
from jax.experimental.pallas import tpu as pltpu

_B = 4096
_P = 1024
_Q = 1024
_E = 8
_TB = 512
_NB = _B // _TB


def _moe_kernel(x_ref, w_ref, b_ref, rw_ref, out_ref, imp_ref, aux_ref):
    i = pl.program_id(0)
    x = x_ref[...]
    logits = jnp.dot(x, rw_ref[...], preferred_element_type=jnp.float32)
    w = jax.nn.softmax(logits, axis=-1)
    acc = jnp.dot(w, b_ref[...], preferred_element_type=jnp.float32)
    for e in range(_E):
        y = jnp.dot(x, w_ref[e], preferred_element_type=jnp.float32)
        acc = acc + w[:, e : e + 1] * y
    out_ref[...] = acc

    part = jnp.sum(w, axis=0, keepdims=True)

    @pl.when(i == 0)
    def _init():
        imp_ref[...] = part

    @pl.when(i > 0)
    def _accum():
        imp_ref[...] = imp_ref[...] + part

    @pl.when(i == _NB - 1)
    def _finalize():
        imp = imp_ref[...] / jnp.float32(_B)
        aux_ref[...] = jnp.float32(_E) * jnp.sum(imp * imp, keepdims=True)


@functools.partial(jax.jit, static_argnames=())
def kernel(inputs, expert_w, expert_b, router_w):
    x16 = inputs.astype(jnp.bfloat16)
    w16 = expert_w.astype(jnp.bfloat16)
    rw16 = router_w.astype(jnp.bfloat16)
    out, _imp, aux = pl.pallas_call(
        _moe_kernel,
        grid=(_NB,),
        in_specs=[
            pl.BlockSpec((_TB, _P), lambda i: (i, 0)),
            pl.BlockSpec((_E, _P, _Q), lambda i: (0, 0, 0)),
            pl.BlockSpec((_E, _Q), lambda i: (0, 0)),
            pl.BlockSpec((_P, _E), lambda i: (0, 0)),
        ],
        out_specs=[
            pl.BlockSpec((_TB, _Q), lambda i: (i, 0)),
            pl.BlockSpec((1, _E), lambda i: (0, 0)),
            pl.BlockSpec((1, 1), lambda i: (0, 0)),
        ],
        out_shape=[
            jax.ShapeDtypeStruct((_B, _Q), jnp.float32),
            jax.ShapeDtypeStruct((1, _E), jnp.float32),
            jax.ShapeDtypeStruct((1, 1), jnp.float32),
        ],
        compiler_params=pltpu.CompilerParams(
            dimension_semantics=("arbitrary",),
        ),
    )(x16, w16, expert_b, rw16)
    return out, aux[0, 0]


# speedup vs baseline: 1.0359x; 1.0359x over previous
import jax
import jax.numpy as jnp
from jax.experimental import pallas as pl
from jax.experimental.pallas import tpu as pltpu

_B = 4096
_P = 1024
_Q = 1024
_E = 8
_QC = 256
_NQ = _Q // _QC


def _moe_kernel(x_ref, w_ref, b_ref, rw_ref, out_ref, aux_ref, wgt_ref):
    q = pl.program_id(0)
    e = pl.program_id(1)

    @pl.when((q == 0) & (e == 0))
    def _router():
        logits = jnp.dot(x_ref[...], rw_ref[...],
                         preferred_element_type=jnp.float32)
        w = jax.nn.softmax(logits, axis=-1)
        wgt_ref[...] = w
        imp = jnp.mean(w, axis=0, keepdims=True)
        aux_ref[...] = jnp.float32(_E) * jnp.sum(imp * imp, keepdims=True)

    w_all = wgt_ref[...]
    mask = jax.lax.broadcasted_iota(jnp.int32, (1, _E), 1) == e
    wcol = jnp.sum(jnp.where(mask, w_all, 0.0), axis=1, keepdims=True)

    y = jnp.dot(x_ref[...], w_ref[0], preferred_element_type=jnp.float32)

    @pl.when(e == 0)
    def _first():
        out_ref[...] = jnp.dot(w_all, b_ref[...],
                               preferred_element_type=jnp.float32) + wcol * y

    @pl.when(e > 0)
    def _accum():
        out_ref[...] = out_ref[...] + wcol * y


def kernel(inputs, expert_w, expert_b, router_w):
    out, aux = pl.pallas_call(
        _moe_kernel,
        grid=(_NQ, _E),
        in_specs=[
            pl.BlockSpec((_B, _P), lambda q, e: (0, 0)),
            pl.BlockSpec((1, _P, _QC), lambda q, e: (e, 0, q)),
            pl.BlockSpec((_E, _QC), lambda q, e: (0, q)),
            pl.BlockSpec((_P, _E), lambda q, e: (0, 0)),
        ],
        out_specs=[
            pl.BlockSpec((_B, _QC), lambda q, e: (0, q)),
            pl.BlockSpec((1, 1), lambda q, e: (0, 0)),
        ],
        out_shape=[
            jax.ShapeDtypeStruct((_B, _Q), jnp.float32),
            jax.ShapeDtypeStruct((1, 1), jnp.float32),
        ],
        scratch_shapes=[pltpu.VMEM((_B, _E), jnp.float32)],
        compiler_params=pltpu.CompilerParams(
            dimension_semantics=("arbitrary", "arbitrary"),
        ),
    )(inputs, expert_w, expert_b, router_w)
    return out, aux[0, 0]


# speedup vs baseline: 1.0860x; 1.0483x over previous
import jax
import jax.numpy as jnp
from jax.experimental import pallas as pl
from jax.experimental.pallas import tpu as pltpu

_B = 4096
_P = 1024
_Q = 1024
_E = 8
_QC = 512
_NQ = _Q // _QC


def _moe_kernel(x_ref, w_ref, b_ref, rw_ref, out_ref, aux_ref, wgt_ref):
    q = pl.program_id(0)
    e = pl.program_id(1)

    @pl.when((q == 0) & (e == 0))
    def _router():
        logits = jnp.dot(x_ref[...], rw_ref[...],
                         preferred_element_type=jnp.float32)
        w = jax.nn.softmax(logits, axis=-1)
        wgt_ref[...] = w
        imp = jnp.mean(w, axis=0, keepdims=True)
        aux_ref[...] = jnp.float32(_E) * jnp.sum(imp * imp, keepdims=True)

    w_all = wgt_ref[...]
    mask = jax.lax.broadcasted_iota(jnp.int32, (1, _E), 1) == e
    wcol = jnp.sum(jnp.where(mask, w_all, 0.0), axis=1, keepdims=True)

    y = jnp.dot(x_ref[...], w_ref[0], preferred_element_type=jnp.float32)

    @pl.when(e == 0)
    def _first():
        out_ref[...] = jnp.dot(w_all, b_ref[...],
                               preferred_element_type=jnp.float32) + wcol * y

    @pl.when(e > 0)
    def _accum():
        out_ref[...] = out_ref[...] + wcol * y


def kernel(inputs, expert_w, expert_b, router_w):
    out, aux = pl.pallas_call(
        _moe_kernel,
        grid=(_NQ, _E),
        in_specs=[
            pl.BlockSpec((_B, _P), lambda q, e: (0, 0)),
            pl.BlockSpec((1, _P, _QC), lambda q, e: (e, 0, q)),
            pl.BlockSpec((_E, _QC), lambda q, e: (0, q)),
            pl.BlockSpec((_P, _E), lambda q, e: (0, 0)),
        ],
        out_specs=[
            pl.BlockSpec((_B, _QC), lambda q, e: (0, q)),
            pl.BlockSpec((1, 1), lambda q, e: (0, 0)),
        ],
        out_shape=[
            jax.ShapeDtypeStruct((_B, _Q), jnp.float32),
            jax.ShapeDtypeStruct((1, 1), jnp.float32),
        ],
        scratch_shapes=[pltpu.VMEM((_B, _E), jnp.float32)],
        compiler_params=pltpu.CompilerParams(
            dimension_semantics=("arbitrary", "arbitrary"),
        ),
    )(inputs, expert_w, expert_b, router_w)
    return out, aux[0, 0]
